# baseline (speedup 1.0000x reference)
import jax
import jax.numpy as jnp
from jax import lax
from jax.experimental import pallas as pl
from jax.experimental.pallas import tpu as pltpu

_B, _N, _H = 8, 128, 768
_NEG_INF = float("-inf")
_BIG = 1 << 30


def _merge_body(set1_ref, set2_ref, out_ref, sim_ref):
    lane_iota = lax.broadcasted_iota(jnp.int32, (_N, _N), 1)
    sub_iota = lax.broadcasted_iota(jnp.int32, (_N, _N), 0)
    ij_iota = sub_iota * _N + lane_iota
    b_iota = lax.broadcasted_iota(jnp.int32, (_B, _N), 0)
    r_iota = lax.broadcasted_iota(jnp.int32, (_B, _N), 1)

    def _rownorm(x):
        xx = x * x
        a = xx[:, 0:128]
        for c in range(1, 6):
            a = a + xx[:, 128 * c:128 * (c + 1)]
        v = a[:, 0:8]
        for g in range(1, 16):
            v = v + a[:, 8 * g:8 * (g + 1)]
        w = v[:, 0:4] + v[:, 4:8]
        t = w[:, 0:2] + w[:, 2:4]
        return jnp.sqrt(t[:, 0:1] + t[:, 1:2])

    for b in range(_B):
        x1 = set1_ref[b]
        x2 = set2_ref[b]
        s1 = x1 / jnp.maximum(_rownorm(x1), 1e-12)
        s2 = x2 / jnp.maximum(_rownorm(x2), 1e-12)
        sim_ref[b] = lax.dot_general(s1, s2, (((1,), (1,)), ((), ())),
                                     preferred_element_type=jnp.float32)

    def step(r, ch):
        for b in range(_B):
            s = sim_ref[b]
            m = jnp.max(s)
            cand = jnp.where(s == m, ij_iota, _BIG)
            chosen = jnp.min(cand)
            sim_ref[b] = jnp.where(ij_iota == chosen, _NEG_INF, s)
            upd = (b_iota == b) & (r_iota == r)
            ch = jnp.where(upd, chosen, ch)
        return ch

    ch = lax.fori_loop(0, _N, step, jnp.zeros((_B, _N), jnp.int32))
    i_idx = ch // _N
    j_idx = ch % _N

    for b in range(_B):
        oht1 = jnp.where(sub_iota == i_idx[b:b + 1, :], jnp.float32(0.5),
                         jnp.float32(0.0))
        oht2 = jnp.where(sub_iota == j_idx[b:b + 1, :], jnp.float32(0.5),
                         jnp.float32(0.0))
        g1 = lax.dot_general(oht1, set1_ref[b], (((0,), (0,)), ((), ())),
                             precision=lax.Precision.HIGHEST,
                             preferred_element_type=jnp.float32)
        g2 = lax.dot_general(oht2, set2_ref[b], (((0,), (0,)), ((), ())),
                             precision=lax.Precision.HIGHEST,
                             preferred_element_type=jnp.float32)
        out_ref[b] = g1 + g2


def _merged_tokens(set1, set2):
    return pl.pallas_call(
        _merge_body,
        out_shape=jax.ShapeDtypeStruct((_B, _N, _H), jnp.float32),
        scratch_shapes=[
            pltpu.VMEM((_B, _N, _N), jnp.float32),
        ],
    )(set1, set2)


def kernel(K):
    batch, num_tokens, hidden = K.shape
    Kr = K.reshape(batch, num_tokens // 2, 2, hidden)
    set1 = Kr[:, :, 0, :]
    set2 = Kr[:, :, 1, :]
    merged = _merged_tokens(set1, set2)
    return (merged, num_tokens // 2)


# speedup vs baseline: 1.3522x; 1.3522x over previous
import jax
import jax.numpy as jnp
from jax import lax
from jax.experimental import pallas as pl
from jax.experimental.pallas import tpu as pltpu

_B, _N, _H = 8, 128, 768
_NEG_INF = float("-inf")
_BIG = 1 << 30


def _merge_body(set1_ref, set2_ref, out_ref, *sim_refs):
    lane_iota = lax.broadcasted_iota(jnp.int32, (_N, _N), 1)
    sub_iota = lax.broadcasted_iota(jnp.int32, (_N, _N), 0)
    ij_iota = sub_iota * _N + lane_iota
    b_iota = lax.broadcasted_iota(jnp.int32, (_B, _N), 0)
    r_iota = lax.broadcasted_iota(jnp.int32, (_B, _N), 1)

    def _rownorm(x):
        xx = x * x
        a = xx[:, 0:128]
        for c in range(1, 6):
            a = a + xx[:, 128 * c:128 * (c + 1)]
        v = a[:, 0:8]
        for g in range(1, 16):
            v = v + a[:, 8 * g:8 * (g + 1)]
        w = v[:, 0:4] + v[:, 4:8]
        t = w[:, 0:2] + w[:, 2:4]
        return jnp.sqrt(t[:, 0:1] + t[:, 1:2])

    for b in range(_B):
        x1 = set1_ref[b]
        x2 = set2_ref[b]
        s1 = x1 / jnp.maximum(_rownorm(x1), 1e-12)
        s2 = x2 / jnp.maximum(_rownorm(x2), 1e-12)
        sim_refs[b][...] = lax.dot_general(s1, s2, (((1,), (1,)), ((), ())),
                                           preferred_element_type=jnp.float32)

    def step(r, ch):
        for b in range(_B):
            s = sim_refs[b][...]
            m = jnp.max(jnp.max(s, axis=1, keepdims=True), axis=0,
                        keepdims=True)
            cand = jnp.where(s == m, ij_iota, _BIG)
            chosen = jnp.min(jnp.min(cand, axis=1, keepdims=True), axis=0,
                             keepdims=True)
            sim_refs[b][...] = jnp.where(ij_iota == chosen, _NEG_INF, s)
            upd = (b_iota == b) & (r_iota == r)
            ch = jnp.where(upd, jnp.broadcast_to(chosen, (_B, _N)), ch)
        return ch

    ch = lax.fori_loop(0, _N, step, jnp.zeros((_B, _N), jnp.int32))
    i_idx = ch // _N
    j_idx = ch % _N

    for b in range(_B):
        oht1 = jnp.where(sub_iota == i_idx[b:b + 1, :], jnp.float32(0.5),
                         jnp.float32(0.0))
        oht2 = jnp.where(sub_iota == j_idx[b:b + 1, :], jnp.float32(0.5),
                         jnp.float32(0.0))
        g1 = lax.dot_general(oht1, set1_ref[b], (((0,), (0,)), ((), ())),
                             precision=lax.Precision.HIGHEST,
                             preferred_element_type=jnp.float32)
        g2 = lax.dot_general(oht2, set2_ref[b], (((0,), (0,)), ((), ())),
                             precision=lax.Precision.HIGHEST,
                             preferred_element_type=jnp.float32)
        out_ref[b] = g1 + g2


def _merged_tokens(set1, set2):
    return pl.pallas_call(
        _merge_body,
        out_shape=jax.ShapeDtypeStruct((_B, _N, _H), jnp.float32),
        scratch_shapes=[
            pltpu.VMEM((_N, _N), jnp.float32) for _ in range(_B)
        ],
    )(set1, set2)


def kernel(K):
    batch, num_tokens, hidden = K.shape
    Kr = K.reshape(batch, num_tokens // 2, 2, hidden)
    set1 = Kr[:, :, 0, :]
    set2 = Kr[:, :, 1, :]
    merged = _merged_tokens(set1, set2)
    return (merged, num_tokens // 2)


# speedup vs baseline: 2.4306x; 1.7975x over previous
import jax
import jax.numpy as jnp
from jax import lax
from jax.experimental import pallas as pl
from jax.experimental.pallas import tpu as pltpu

_B, _N, _H = 8, 128, 768
_NN = _N * _N
_NEG_INF = float("-inf")
_BIG = 1 << 30


def _merge_body(set1_ref, set2_ref, out_ref, flat_ref):
    sub_iota = lax.broadcasted_iota(jnp.int32, (_N, _N), 0)
    flat_iota = lax.broadcasted_iota(jnp.int32, (_B, _NN), 1)
    r_iota = lax.broadcasted_iota(jnp.int32, (_B, _N), 1)

    def _rownorm(x):
        xx = x * x
        a = xx[:, 0:128]
        for c in range(1, 6):
            a = a + xx[:, 128 * c:128 * (c + 1)]
        v = a[:, 0:8]
        for g in range(1, 16):
            v = v + a[:, 8 * g:8 * (g + 1)]
        w = v[:, 0:4] + v[:, 4:8]
        t = w[:, 0:2] + w[:, 2:4]
        return jnp.sqrt(t[:, 0:1] + t[:, 1:2])

    for b in range(_B):
        x1 = set1_ref[b]
        x2 = set2_ref[b]
        s1 = x1 / jnp.maximum(_rownorm(x1), 1e-12)
        s2 = x2 / jnp.maximum(_rownorm(x2), 1e-12)
        sim_b = lax.dot_general(s1, s2, (((1,), (1,)), ((), ())),
                                preferred_element_type=jnp.float32)
        flat_ref[b:b + 1, :] = sim_b.reshape(1, _NN)

    def step(r, ch):
        f = flat_ref[...]
        m = jnp.max(f, axis=1, keepdims=True)
        cand = jnp.where(f == m, flat_iota, _BIG)
        chosen = jnp.min(cand, axis=1, keepdims=True)
        flat_ref[...] = jnp.where(flat_iota == chosen, _NEG_INF, f)
        return jnp.where(r_iota == r, jnp.broadcast_to(chosen, (_B, _N)), ch)

    ch = lax.fori_loop(0, _N, step, jnp.zeros((_B, _N), jnp.int32))
    i_idx = ch // _N
    j_idx = ch % _N

    for b in range(_B):
        oht1 = jnp.where(sub_iota == i_idx[b:b + 1, :], jnp.float32(0.5),
                         jnp.float32(0.0))
        oht2 = jnp.where(sub_iota == j_idx[b:b + 1, :], jnp.float32(0.5),
                         jnp.float32(0.0))
        g1 = lax.dot_general(oht1, set1_ref[b], (((0,), (0,)), ((), ())),
                             precision=lax.Precision.HIGHEST,
                             preferred_element_type=jnp.float32)
        g2 = lax.dot_general(oht2, set2_ref[b], (((0,), (0,)), ((), ())),
                             precision=lax.Precision.HIGHEST,
                             preferred_element_type=jnp.float32)
        out_ref[b] = g1 + g2


def _merged_tokens(set1, set2):
    return pl.pallas_call(
        _merge_body,
        out_shape=jax.ShapeDtypeStruct((_B, _N, _H), jnp.float32),
        scratch_shapes=[
            pltpu.VMEM((_B, _NN), jnp.float32),
        ],
    )(set1, set2)


def kernel(K):
    batch, num_tokens, hidden = K.shape
    Kr = K.reshape(batch, num_tokens // 2, 2, hidden)
    set1 = Kr[:, :, 0, :]
    set2 = Kr[:, :, 1, :]
    merged = _merged_tokens(set1, set2)
    return (merged, num_tokens // 2)


# speedup vs baseline: 3.6895x; 1.5179x over previous
import functools

import jax
import jax.numpy as jnp
from jax import lax
from jax.experimental import pallas as pl
from jax.experimental.pallas import tpu as pltpu
from jax.experimental.pallas import tpu_sc as plsc

_B, _N, _H = 8, 128, 768
_NN = _N * _N
_NEG_INF = float("-inf")


def _sim_body(set1_ref, set2_ref, flat_ref):
    def _rownorm(x):
        xx = x * x
        a = xx[:, 0:128]
        for c in range(1, 6):
            a = a + xx[:, 128 * c:128 * (c + 1)]
        v = a[:, 0:8]
        for g in range(1, 16):
            v = v + a[:, 8 * g:8 * (g + 1)]
        w = v[:, 0:4] + v[:, 4:8]
        t = w[:, 0:2] + w[:, 2:4]
        return jnp.sqrt(t[:, 0:1] + t[:, 1:2])

    for b in range(_B):
        x1 = set1_ref[b]
        x2 = set2_ref[b]
        s1 = x1 / jnp.maximum(_rownorm(x1), 1e-12)
        s2 = x2 / jnp.maximum(_rownorm(x2), 1e-12)
        sim_b = lax.dot_general(s1, s2, (((1,), (1,)), ((), ())),
                                preferred_element_type=jnp.float32)
        flat_ref[b:b + 1, :] = sim_b.reshape(1, _NN)


def _similarity(set1, set2):
    return pl.pallas_call(
        _sim_body,
        out_shape=jax.ShapeDtypeStruct((_B, _NN), jnp.float32),
    )(set1, set2)


def _topk_sc(flat):
    mesh = plsc.VectorSubcoreMesh(core_axis_name="c", subcore_axis_name="s")

    @functools.partial(
        pl.kernel, mesh=mesh,
        out_type=jax.ShapeDtypeStruct((_B, _N), jnp.int32),
        scratch_types=[
            pltpu.VMEM((_NN,), jnp.float32),
            pltpu.VMEM((_N,), jnp.float32),
            pltpu.VMEM((_N,), jnp.int32),
        ],
    )
    def k(flat_hbm, out_hbm, buf, rmax, ch):
        wid = lax.axis_index("s") * 2 + lax.axis_index("c")
        lane = lax.iota(jnp.int32, 16)

        def _smax(v):
            w = jnp.maximum(v, jnp.flip(v))
            s = w[0]
            for kk in range(1, 8):
                s = jnp.maximum(s, w[kk])
            return s

        def _smin(v):
            w = jnp.minimum(v, jnp.flip(v))
            s = w[0]
            for kk in range(1, 8):
                s = jnp.minimum(s, w[kk])
            return s

        @pl.when(wid < _B)
        def _():
            pltpu.sync_copy(flat_hbm.at[wid], buf)

            def seed_row(i, carry):
                acc = buf[pl.ds(i * _N, 16)]
                for q in range(1, 8):
                    acc = jnp.maximum(acc, buf[pl.ds(i * _N + 16 * q, 16)])
                rm = _smax(acc)
                c = (i // 16) * 16
                chunk = rmax[pl.ds(c, 16)]
                rmax[pl.ds(c, 16)] = jnp.where(lane == i - c, rm, chunk)
                return carry

            lax.fori_loop(0, _N, seed_row, 0)

            def step(r, carry):
                rvs = [rmax[pl.ds(16 * q, 16)] for q in range(8)]
                mr = rvs[0]
                for v in rvs[1:]:
                    mr = jnp.maximum(mr, v)
                m = _smax(mr)
                icand = [jnp.where(rv == m, lane + 16 * q, _NN)
                         for q, rv in enumerate(rvs)]
                imin = icand[0]
                for v in icand[1:]:
                    imin = jnp.minimum(imin, v)
                i_star = _smin(imin)
                base = i_star * _N

                row = [buf[pl.ds(base + 16 * q, 16)] for q in range(8)]
                jcand = [jnp.where(rq == m, lane + 16 * q, _NN)
                         for q, rq in enumerate(row)]
                jmin = jcand[0]
                for v in jcand[1:]:
                    jmin = jnp.minimum(jmin, v)
                j_star = _smin(jmin)

                jc = (j_star // 16) * 16
                chunk = buf[pl.ds(base + jc, 16)]
                buf[pl.ds(base + jc, 16)] = jnp.where(lane == j_star - jc,
                                                      _NEG_INF, chunk)
                masked = [jnp.where(lane + 16 * q == j_star, _NEG_INF, rq)
                          for q, rq in enumerate(row)]
                nmr = masked[0]
                for v in masked[1:]:
                    nmr = jnp.maximum(nmr, v)
                nm = _smax(nmr)
                ic = (i_star // 16) * 16
                rchunk = rmax[pl.ds(ic, 16)]
                rmax[pl.ds(ic, 16)] = jnp.where(lane == i_star - ic, nm,
                                                rchunk)
                rc = (r // 16) * 16
                cchunk = ch[pl.ds(rc, 16)]
                ch[pl.ds(rc, 16)] = jnp.where(lane == r - rc, base + j_star,
                                              cchunk)
                return carry

            lax.fori_loop(0, _N, step, 0)
            pltpu.sync_copy(ch, out_hbm.at[wid])

    return k(flat)


def _gather_body(set1_ref, set2_ref, ch_ref, out_ref):
    sub_iota = lax.broadcasted_iota(jnp.int32, (_N, _N), 0)
    ch = ch_ref[...]
    i_idx = ch // _N
    j_idx = ch % _N
    for b in range(_B):
        oht1 = jnp.where(sub_iota == i_idx[b:b + 1, :], jnp.float32(0.5),
                         jnp.float32(0.0))
        oht2 = jnp.where(sub_iota == j_idx[b:b + 1, :], jnp.float32(0.5),
                         jnp.float32(0.0))
        g1 = lax.dot_general(oht1, set1_ref[b], (((0,), (0,)), ((), ())),
                             precision=lax.Precision.HIGHEST,
                             preferred_element_type=jnp.float32)
        g2 = lax.dot_general(oht2, set2_ref[b], (((0,), (0,)), ((), ())),
                             precision=lax.Precision.HIGHEST,
                             preferred_element_type=jnp.float32)
        out_ref[b] = g1 + g2


def _gather_avg(set1, set2, ch):
    return pl.pallas_call(
        _gather_body,
        out_shape=jax.ShapeDtypeStruct((_B, _N, _H), jnp.float32),
    )(set1, set2, ch)


def kernel(K):
    batch, num_tokens, hidden = K.shape
    Kr = K.reshape(batch, num_tokens // 2, 2, hidden)
    set1 = Kr[:, :, 0, :]
    set2 = Kr[:, :, 1, :]
    flat = _similarity(set1, set2)
    ch = _topk_sc(flat)
    merged = _gather_avg(set1, set2, ch)
    return (merged, num_tokens // 2)
